# baseline (speedup 1.0000x reference)
import functools

import jax
import jax.numpy as jnp
from jax import lax
from jax.experimental import pallas as pl
from jax.experimental.pallas import tpu as pltpu
from jax.experimental.pallas import tpu_sc as plsc

_NUM_WORKERS = 32
_R = 20
_B = 16384 * 20
_D = 64
_ROWS_W = 16384 // _NUM_WORKERS
_PER_W = _ROWS_W * _R
_GPS = 8
_SLOT = _GPS * _R
_NSLOT = _PER_W // _SLOT
_NBUF = 4
_K = 2
_S = _NSLOT // _NBUF


@functools.partial(
    pl.kernel,
    out_type=jax.ShapeDtypeStruct((_B, _D), jnp.float32),
    mesh=plsc.VectorSubcoreMesh(core_axis_name="c", subcore_axis_name="s"),
    scratch_types=[
        pltpu.VMEM((_ROWS_W, _R), jnp.int32),
        pltpu.VMEM((_NBUF, _SLOT, 128), jnp.float32),
        pltpu.SemaphoreType.DMA((_NBUF,)),
        pltpu.SemaphoreType.DMA((_NBUF,)),
    ],
    compiler_params=pltpu.CompilerParams(use_tc_tiling_on_sc=False),
)
def _gather_kernel(idx_hbm, table_hbm, out_hbm, idx_v, bufs, gsem, osem):
    wid = lax.axis_index("s") * 2 + lax.axis_index("c")
    base = wid * _PER_W
    pltpu.sync_copy(idx_hbm.at[pl.ds(wid * _ROWS_W, _ROWS_W)], idx_v)

    def out_slice(c):
        return out_hbm.at[pl.ds(base + c * _SLOT, _SLOT)]

    def fire_gathers(g, b):
        @pl.loop(0, _GPS)
        def _g(q):
            pltpu.async_copy(table_hbm.at[idx_v.at[g * _GPS + q]],
                             bufs.at[b, pl.ds(q * _R, _R)], gsem.at[b])

    def wait_gathers(b):
        pltpu.make_async_copy(table_hbm.at[pl.ds(0, _SLOT)], bufs.at[b],
                              gsem.at[b]).wait()

    @pl.loop(0, _S)
    def _super(s):
        for b in range(_NBUF):
            g = s * _NBUF + b

            @pl.when(s > 0)
            def _():
                pltpu.make_async_copy(bufs.at[b, pl.ds(0, _SLOT), pl.ds(0, _D)],
                                      out_slice(g - _NBUF), osem.at[b]).wait()

            fire_gathers(g, b)

            bk = (b - _K) % _NBUF
            c = g - _K

            @pl.when(c >= 0)
            def _():
                wait_gathers(bk)
                pltpu.async_copy(bufs.at[bk, pl.ds(0, _SLOT), pl.ds(0, _D)],
                                 out_slice(c), osem.at[bk])

    for c in range(_NSLOT - _K, _NSLOT):
        b = c % _NBUF
        wait_gathers(b)
        pltpu.async_copy(bufs.at[b, pl.ds(0, _SLOT), pl.ds(0, _D)],
                         out_slice(c), osem.at[b])
    for c in range(_NSLOT - _NBUF, _NSLOT):
        b = c % _NBUF
        pltpu.make_async_copy(bufs.at[b, pl.ds(0, _SLOT), pl.ds(0, _D)],
                              out_slice(c), osem.at[b]).wait()


_TC_CB = 512
_TC_GRID = (1000000 + _TC_CB - 1) // _TC_CB
_NROW2 = _TC_GRID * _TC_CB


def _transpose_body(wt_ref, out_ref):
    y = jax.lax.dot_general(wt_ref[...], jnp.eye(_D, dtype=jnp.float32),
                            dimension_numbers=(((0,), (0,)), ((), ())),
                            preferred_element_type=jnp.float32)
    out_ref[...] = jnp.concatenate([y, y], axis=1)


def _detile_table(weight):
    wt = weight.T
    return pl.pallas_call(
        _transpose_body,
        grid=(_TC_GRID,),
        in_specs=[pl.BlockSpec((_D, _TC_CB), lambda i: (0, i))],
        out_specs=pl.BlockSpec((_TC_CB, 128), lambda i: (i, 0)),
        out_shape=jax.ShapeDtypeStruct((_NROW2, 128), jnp.float32),
    )(wt)


def kernel(embed_id, weight):
    w2 = _detile_table(weight)
    out = _gather_kernel(embed_id, w2)
    return out.reshape(16384, 20, _D)


# speedup vs baseline: 1.9525x; 1.9525x over previous
import functools

import jax
import jax.numpy as jnp
from jax import lax
from jax.experimental import pallas as pl
from jax.experimental.pallas import tpu as pltpu
from jax.experimental.pallas import tpu_sc as plsc

_NUM_WORKERS = 32
_R = 20
_B = 16384 * 20
_D = 64
_ROWS_W = 16384 // _NUM_WORKERS
_PER_W = _ROWS_W * _R
_GPS = 8
_SLOT = _GPS * _R
_NSLOT = _PER_W // _SLOT
_NBUF = 4
_K = 2
_S = _NSLOT // _NBUF


@functools.partial(
    pl.kernel,
    out_type=jax.ShapeDtypeStruct((_B, _D), jnp.float32),
    mesh=plsc.VectorSubcoreMesh(core_axis_name="c", subcore_axis_name="s"),
    scratch_types=[
        pltpu.VMEM((_ROWS_W, _R), jnp.int32),
        pltpu.VMEM((_NBUF, _SLOT, 128), jnp.float32),
        pltpu.SemaphoreType.DMA((_NBUF,)),
        pltpu.SemaphoreType.DMA((_NBUF,)),
    ],
    compiler_params=pltpu.CompilerParams(use_tc_tiling_on_sc=False),
)
def _gather_kernel(idx_hbm, table_hbm, out_hbm, idx_v, bufs, gsem, osem):
    wid = lax.axis_index("s") * 2 + lax.axis_index("c")
    base = wid * _PER_W
    pltpu.sync_copy(idx_hbm.at[pl.ds(wid * _ROWS_W, _ROWS_W)], idx_v)

    def out_slice(c):
        return out_hbm.at[pl.ds(base + c * _SLOT, _SLOT)]

    def fire_gathers(g, b):
        @pl.loop(0, _GPS)
        def _g(q):
            pltpu.async_copy(table_hbm.at[idx_v.at[g * _GPS + q]],
                             bufs.at[b, pl.ds(q * _R, _R)], gsem.at[b])

    def wait_gathers(b):
        pltpu.make_async_copy(table_hbm.at[pl.ds(0, _SLOT)], bufs.at[b],
                              gsem.at[b]).wait()

    @pl.loop(0, _S)
    def _super(s):
        for b in range(_NBUF):
            g = s * _NBUF + b

            @pl.when(s > 0)
            def _():
                pltpu.make_async_copy(bufs.at[b, pl.ds(0, _SLOT), pl.ds(0, _D)],
                                      out_slice(g - _NBUF), osem.at[b]).wait()

            fire_gathers(g, b)

            bk = (b - _K) % _NBUF
            c = g - _K

            @pl.when(c >= 0)
            def _():
                wait_gathers(bk)
                pltpu.async_copy(bufs.at[bk, pl.ds(0, _SLOT), pl.ds(0, _D)],
                                 out_slice(c), osem.at[bk])

    for c in range(_NSLOT - _K, _NSLOT):
        b = c % _NBUF
        wait_gathers(b)
        pltpu.async_copy(bufs.at[b, pl.ds(0, _SLOT), pl.ds(0, _D)],
                         out_slice(c), osem.at[b])
    for c in range(_NSLOT - _NBUF, _NSLOT):
        b = c % _NBUF
        pltpu.make_async_copy(bufs.at[b, pl.ds(0, _SLOT), pl.ds(0, _D)],
                              out_slice(c), osem.at[b]).wait()


_TC_CB = 2048
_TC_GRID = (1000000 + _TC_CB - 1) // _TC_CB
_NROW2 = _TC_GRID * _TC_CB


def _transpose_body(wt_ref, out_ref):
    y = jax.lax.dot_general(wt_ref[...], jnp.eye(_D, dtype=jnp.float32),
                            dimension_numbers=(((0,), (0,)), ((), ())),
                            preferred_element_type=jnp.float32)
    out_ref[...] = jnp.concatenate([y, y], axis=1)


def _detile_table(weight):
    wt = weight.T
    return pl.pallas_call(
        _transpose_body,
        grid=(_TC_GRID,),
        in_specs=[pl.BlockSpec((_D, _TC_CB), lambda i: (0, i))],
        out_specs=pl.BlockSpec((_TC_CB, 128), lambda i: (i, 0)),
        out_shape=jax.ShapeDtypeStruct((_NROW2, 128), jnp.float32),
    )(wt)


def kernel(embed_id, weight):
    w2 = _detile_table(weight)
    out = _gather_kernel(embed_id, w2)
    return out.reshape(16384, 20, _D)


# speedup vs baseline: 2.5255x; 1.2935x over previous
import functools

import jax
import jax.numpy as jnp
from jax import lax
from jax.experimental import pallas as pl
from jax.experimental.pallas import tpu as pltpu
from jax.experimental.pallas import tpu_sc as plsc

_NUM_WORKERS = 32
_R = 20
_B = 16384 * 20
_D = 64
_ROWS_W = 16384 // _NUM_WORKERS
_PER_W = _ROWS_W * _R
_GPS = 8
_SLOT = _GPS * _R
_NSLOT = _PER_W // _SLOT
_NBUF = 4
_K = 2
_S = _NSLOT // _NBUF


@functools.partial(
    pl.kernel,
    out_type=jax.ShapeDtypeStruct((_B, _D), jnp.float32),
    mesh=plsc.VectorSubcoreMesh(core_axis_name="c", subcore_axis_name="s"),
    scratch_types=[
        pltpu.VMEM((_ROWS_W, _R), jnp.int32),
        pltpu.VMEM((_NBUF, _SLOT, 128), jnp.float32),
        pltpu.SemaphoreType.DMA((_NBUF,)),
        pltpu.SemaphoreType.DMA((_NBUF,)),
    ],
    compiler_params=pltpu.CompilerParams(use_tc_tiling_on_sc=False),
)
def _gather_kernel(idx_hbm, table_hbm, out_hbm, idx_v, bufs, gsem, osem):
    wid = lax.axis_index("s") * 2 + lax.axis_index("c")
    base = wid * _PER_W
    pltpu.sync_copy(idx_hbm.at[pl.ds(wid * _ROWS_W, _ROWS_W)], idx_v)

    def out_slice(c):
        return out_hbm.at[pl.ds(base + c * _SLOT, _SLOT)]

    def fire_gathers(g, b):
        @pl.loop(0, _GPS)
        def _g(q):
            pltpu.async_copy(table_hbm.at[idx_v.at[g * _GPS + q]],
                             bufs.at[b, pl.ds(q * _R, _R)], gsem.at[b])

    def wait_gathers(b):
        pltpu.make_async_copy(table_hbm.at[pl.ds(0, _SLOT)], bufs.at[b],
                              gsem.at[b]).wait()

    @pl.loop(0, _S)
    def _super(s):
        for b in range(_NBUF):
            g = s * _NBUF + b

            @pl.when(s > 0)
            def _():
                pltpu.make_async_copy(bufs.at[b, pl.ds(0, _SLOT), pl.ds(0, _D)],
                                      out_slice(g - _NBUF), osem.at[b]).wait()

            fire_gathers(g, b)

            bk = (b - _K) % _NBUF
            c = g - _K

            @pl.when(c >= 0)
            def _():
                wait_gathers(bk)
                pltpu.async_copy(bufs.at[bk, pl.ds(0, _SLOT), pl.ds(0, _D)],
                                 out_slice(c), osem.at[bk])

    for c in range(_NSLOT - _K, _NSLOT):
        b = c % _NBUF
        wait_gathers(b)
        pltpu.async_copy(bufs.at[b, pl.ds(0, _SLOT), pl.ds(0, _D)],
                         out_slice(c), osem.at[b])
    for c in range(_NSLOT - _NBUF, _NSLOT):
        b = c % _NBUF
        pltpu.make_async_copy(bufs.at[b, pl.ds(0, _SLOT), pl.ds(0, _D)],
                              out_slice(c), osem.at[b]).wait()


_TC_CB = 8192
_TC_GRID = (1000000 + _TC_CB - 1) // _TC_CB
_NROW2 = _TC_GRID * _TC_CB


def _transpose_body(wt_ref, out_ref):
    y = jax.lax.dot_general(wt_ref[...], jnp.eye(_D, dtype=jnp.float32),
                            dimension_numbers=(((0,), (0,)), ((), ())),
                            preferred_element_type=jnp.float32)
    out_ref[...] = jnp.concatenate([y, y], axis=1)


def _detile_table(weight):
    wt = weight.T
    return pl.pallas_call(
        _transpose_body,
        grid=(_TC_GRID,),
        in_specs=[pl.BlockSpec((_D, _TC_CB), lambda i: (0, i))],
        out_specs=pl.BlockSpec((_TC_CB, 128), lambda i: (i, 0)),
        out_shape=jax.ShapeDtypeStruct((_NROW2, 128), jnp.float32),
    )(wt)


def kernel(embed_id, weight):
    w2 = _detile_table(weight)
    out = _gather_kernel(embed_id, w2)
    return out.reshape(16384, 20, _D)


# speedup vs baseline: 2.6597x; 1.0532x over previous
import functools

import jax
import jax.numpy as jnp
from jax import lax
from jax.experimental import pallas as pl
from jax.experimental.pallas import tpu as pltpu
from jax.experimental.pallas import tpu_sc as plsc

_NUM_WORKERS = 32
_R = 20
_B = 16384 * 20
_D = 64
_ROWS_W = 16384 // _NUM_WORKERS
_PER_W = _ROWS_W * _R
_GPS = 8
_SLOT = _GPS * _R
_NSLOT = _PER_W // _SLOT
_NBUF = 4
_K = 2
_S = _NSLOT // _NBUF


@functools.partial(
    pl.kernel,
    out_type=jax.ShapeDtypeStruct((_B, _D), jnp.float32),
    mesh=plsc.VectorSubcoreMesh(core_axis_name="c", subcore_axis_name="s"),
    scratch_types=[
        pltpu.VMEM((_ROWS_W, _R), jnp.int32),
        pltpu.VMEM((_NBUF, _SLOT, 128), jnp.float32),
        pltpu.SemaphoreType.DMA((_NBUF,)),
        pltpu.SemaphoreType.DMA((_NBUF,)),
    ],
    compiler_params=pltpu.CompilerParams(use_tc_tiling_on_sc=False),
)
def _gather_kernel(idx_hbm, table_hbm, out_hbm, idx_v, bufs, gsem, osem):
    wid = lax.axis_index("s") * 2 + lax.axis_index("c")
    base = wid * _PER_W
    pltpu.sync_copy(idx_hbm.at[pl.ds(wid * _ROWS_W, _ROWS_W)], idx_v)

    def out_slice(c):
        return out_hbm.at[pl.ds(base + c * _SLOT, _SLOT)]

    def fire_gathers(g, b):
        @pl.loop(0, _GPS)
        def _g(q):
            pltpu.async_copy(table_hbm.at[idx_v.at[g * _GPS + q]],
                             bufs.at[b, pl.ds(q * _R, _R)], gsem.at[b])

    def wait_gathers(b):
        pltpu.make_async_copy(table_hbm.at[pl.ds(0, _SLOT)], bufs.at[b],
                              gsem.at[b]).wait()

    @pl.loop(0, _S)
    def _super(s):
        for b in range(_NBUF):
            g = s * _NBUF + b

            @pl.when(s > 0)
            def _():
                pltpu.make_async_copy(bufs.at[b, pl.ds(0, _SLOT), pl.ds(0, _D)],
                                      out_slice(g - _NBUF), osem.at[b]).wait()

            fire_gathers(g, b)

            bk = (b - _K) % _NBUF
            c = g - _K

            @pl.when(c >= 0)
            def _():
                wait_gathers(bk)
                pltpu.async_copy(bufs.at[bk, pl.ds(0, _SLOT), pl.ds(0, _D)],
                                 out_slice(c), osem.at[bk])

    for c in range(_NSLOT - _K, _NSLOT):
        b = c % _NBUF
        wait_gathers(b)
        pltpu.async_copy(bufs.at[b, pl.ds(0, _SLOT), pl.ds(0, _D)],
                         out_slice(c), osem.at[b])
    for c in range(_NSLOT - _NBUF, _NSLOT):
        b = c % _NBUF
        pltpu.make_async_copy(bufs.at[b, pl.ds(0, _SLOT), pl.ds(0, _D)],
                              out_slice(c), osem.at[b]).wait()


_TC_CB = 16384
_TC_GRID = (1000000 + _TC_CB - 1) // _TC_CB
_NROW2 = _TC_GRID * _TC_CB


def _transpose_body(wt_ref, out_ref):
    y = jax.lax.dot_general(wt_ref[...], jnp.eye(_D, dtype=jnp.float32),
                            dimension_numbers=(((0,), (0,)), ((), ())),
                            preferred_element_type=jnp.float32)
    out_ref[...] = jnp.concatenate([y, y], axis=1)


def _detile_table(weight):
    wt = weight.T
    return pl.pallas_call(
        _transpose_body,
        grid=(_TC_GRID,),
        in_specs=[pl.BlockSpec((_D, _TC_CB), lambda i: (0, i))],
        out_specs=pl.BlockSpec((_TC_CB, 128), lambda i: (i, 0)),
        out_shape=jax.ShapeDtypeStruct((_NROW2, 128), jnp.float32),
    )(wt)


def kernel(embed_id, weight):
    w2 = _detile_table(weight)
    out = _gather_kernel(embed_id, w2)
    return out.reshape(16384, 20, _D)


# speedup vs baseline: 2.6605x; 1.0003x over previous
import functools

import jax
import jax.numpy as jnp
from jax import lax
from jax.experimental import pallas as pl
from jax.experimental.pallas import tpu as pltpu
from jax.experimental.pallas import tpu_sc as plsc

_NUM_WORKERS = 32
_R = 20
_B = 16384 * 20
_D = 64
_ROWS_W = 16384 // _NUM_WORKERS
_PER_W = _ROWS_W * _R
_GPS = 8
_SLOT = _GPS * _R
_NSLOT = _PER_W // _SLOT
_NBUF = 4
_K = 2
_S = _NSLOT // _NBUF


@functools.partial(
    pl.kernel,
    out_type=jax.ShapeDtypeStruct((_B, _D), jnp.float32),
    mesh=plsc.VectorSubcoreMesh(core_axis_name="c", subcore_axis_name="s"),
    scratch_types=[
        pltpu.VMEM((_ROWS_W, _R), jnp.int32),
        pltpu.VMEM((_NBUF, _SLOT, 128), jnp.float32),
        pltpu.SemaphoreType.DMA((_NBUF,)),
        pltpu.SemaphoreType.DMA((_NBUF,)),
    ],
    compiler_params=pltpu.CompilerParams(use_tc_tiling_on_sc=False),
)
def _gather_kernel(idx_hbm, table_hbm, out_hbm, idx_v, bufs, gsem, osem):
    wid = lax.axis_index("s") * 2 + lax.axis_index("c")
    base = wid * _PER_W
    pltpu.sync_copy(idx_hbm.at[pl.ds(wid * _ROWS_W, _ROWS_W)], idx_v)

    def out_slice(c):
        return out_hbm.at[pl.ds(base + c * _SLOT, _SLOT)]

    def fire_gathers(g, b):
        @pl.loop(0, _GPS)
        def _g(q):
            pltpu.async_copy(table_hbm.at[idx_v.at[g * _GPS + q]],
                             bufs.at[b, pl.ds(q * _R, _R)], gsem.at[b])

    def wait_gathers(b):
        pltpu.make_async_copy(table_hbm.at[pl.ds(0, _SLOT)], bufs.at[b],
                              gsem.at[b]).wait()

    @pl.loop(0, _S)
    def _super(s):
        for b in range(_NBUF):
            g = s * _NBUF + b

            @pl.when(s > 0)
            def _():
                pltpu.make_async_copy(bufs.at[b, pl.ds(0, _SLOT), pl.ds(0, _D)],
                                      out_slice(g - _NBUF), osem.at[b]).wait()

            fire_gathers(g, b)

            bk = (b - _K) % _NBUF
            c = g - _K

            @pl.when(c >= 0)
            def _():
                wait_gathers(bk)
                pltpu.async_copy(bufs.at[bk, pl.ds(0, _SLOT), pl.ds(0, _D)],
                                 out_slice(c), osem.at[bk])

    for c in range(_NSLOT - _K, _NSLOT):
        b = c % _NBUF
        wait_gathers(b)
        pltpu.async_copy(bufs.at[b, pl.ds(0, _SLOT), pl.ds(0, _D)],
                         out_slice(c), osem.at[b])
    for c in range(_NSLOT - _NBUF, _NSLOT):
        b = c % _NBUF
        pltpu.make_async_copy(bufs.at[b, pl.ds(0, _SLOT), pl.ds(0, _D)],
                              out_slice(c), osem.at[b]).wait()


_TC_CB = 16384
_TC_GRID = (1000000 + _TC_CB - 1) // _TC_CB
_NROW2 = _TC_GRID * _TC_CB


def _transpose_body(wt_ref, out_ref):
    y = wt_ref[...].T
    out_ref[...] = jnp.concatenate([y, y], axis=1)


def _detile_table(weight):
    wt = weight.T
    return pl.pallas_call(
        _transpose_body,
        grid=(_TC_GRID,),
        in_specs=[pl.BlockSpec((_D, _TC_CB), lambda i: (0, i))],
        out_specs=pl.BlockSpec((_TC_CB, 128), lambda i: (i, 0)),
        out_shape=jax.ShapeDtypeStruct((_NROW2, 128), jnp.float32),
    )(wt)


def kernel(embed_id, weight):
    w2 = _detile_table(weight)
    out = _gather_kernel(embed_id, w2)
    return out.reshape(16384, 20, _D)
